# baseline (speedup 1.0000x reference)
import jax
import jax.numpy as jnp
from jax import lax
from jax.experimental import pallas as pl
from jax.experimental.pallas import tpu as pltpu
from jax.experimental.pallas import tpu_sc as plsc

_B, _S, _D, _T = 4, 4096, 2048, 64
_N = _T + _S

_CR = 256
_NCH = _S // _CR
_NBUF = 8
_LAG = 4

_NC, _NS = 2, 16


def _tc_body(text, task, oe, buf, isem, osem):
    chunks = []
    for b in range(_B):
        chunks.append((task, b, 0, 0, _T))
        for j in range(_NCH):
            chunks.append((text, b, j * _CR, _T + j * _CR, _CR))

    n = len(chunks)

    def in_copy(k):
        src, sb, sr, dr, rows = chunks[k]
        return pltpu.make_async_copy(
            src.at[sb, pl.ds(sr, rows), :],
            buf.at[k % _NBUF, pl.ds(0, rows), :], isem.at[k % _NBUF])

    def out_copy(k):
        src, sb, sr, dr, rows = chunks[k]
        return pltpu.make_async_copy(
            buf.at[k % _NBUF, pl.ds(0, rows), :],
            oe.at[sb, pl.ds(dr, rows), :], osem.at[k % _NBUF])

    for k in range(n + _LAG):
        if k < n:
            if k >= _NBUF:
                out_copy(k - _NBUF).wait()
            in_copy(k).start()
        if k >= _LAG:
            j = k - _LAG
            in_copy(j).wait()
            out_copy(j).start()

    for k in range(n - _NBUF, n):
        out_copy(k).wait()


def _sc_body(mask, labels, om, ol, mbuf, lbuf, pbuf_m, pbuf_l):
    w = lax.axis_index("c") * _NS + lax.axis_index("s")

    @pl.when(w < _B)
    def _mask_labels():
        b = w
        for q in range(_T // 16):
            pbuf_m[pl.ds(q * 16, 16)] = jnp.ones((16,), jnp.int32)
            pbuf_l[pl.ds(q * 16, 16)] = jnp.full((16,), -100, jnp.int32)
        mrow = pl.multiple_of(b * _S, 8)
        orow = pl.multiple_of(b * _N, 8)
        orow_t = pl.multiple_of(b * _N + _T, 8)
        pltpu.sync_copy(pbuf_m, om.at[pl.ds(orow, _T)])
        pltpu.sync_copy(pbuf_l, ol.at[pl.ds(orow, _T)])
        pltpu.sync_copy(mask.at[pl.ds(mrow, _S)], mbuf)
        pltpu.sync_copy(mbuf, om.at[pl.ds(orow_t, _S)])
        pltpu.sync_copy(labels.at[pl.ds(mrow, _S)], lbuf)
        pltpu.sync_copy(lbuf, ol.at[pl.ds(orow_t, _S)])


@jax.jit
def _inject(text_embeds, attention_mask, labels, task_embeds):
    any_spec = pl.BlockSpec(memory_space=pl.ANY)

    mesh = plsc.VectorSubcoreMesh(core_axis_name="c", subcore_axis_name="s",
                                  num_cores=_NC, num_subcores=_NS)
    om, ol = pl.kernel(
        _sc_body,
        out_type=(
            jax.ShapeDtypeStruct((_B * _N,), jnp.int32),
            jax.ShapeDtypeStruct((_B * _N,), jnp.int32),
        ),
        mesh=mesh,
        scratch_types=(
            pltpu.VMEM((_S,), jnp.int32),
            pltpu.VMEM((_S,), jnp.int32),
            pltpu.VMEM((_T,), jnp.int32),
            pltpu.VMEM((_T,), jnp.int32),
        ),
    )(attention_mask.reshape(-1), labels.reshape(-1))

    oe = pl.pallas_call(
        _tc_body,
        in_specs=[any_spec, any_spec],
        out_specs=any_spec,
        out_shape=jax.ShapeDtypeStruct((_B, _N, _D), jnp.float32),
        scratch_shapes=[
            pltpu.VMEM((_NBUF, _CR, _D), jnp.float32),
            pltpu.SemaphoreType.DMA((_NBUF,)),
            pltpu.SemaphoreType.DMA((_NBUF,)),
        ],
    )(text_embeds, task_embeds)

    return oe, om.reshape(_B, _N), ol.reshape(_B, _N)


def kernel(text_embeds, attention_mask, labels, task_embeds):
    b, s, d = text_embeds.shape
    t = task_embeds.shape[1]
    assert (b, s, d, t) == (_B, _S, _D, _T)
    return _inject(text_embeds, attention_mask, labels, task_embeds)


# speedup vs baseline: 1.2299x; 1.2299x over previous
import jax
import jax.numpy as jnp
from jax.experimental import pallas as pl
from jax.experimental.pallas import tpu as pltpu

_B, _S, _D, _T = 4, 4096, 2048, 64
_N = _T + _S
_CR = 512
_NCH = _S // _CR
_NBUF = 8
_LAG = 4


def _body(text, mask, labels, task, oe, om, ol, buf, isem, osem):
    chunks = []
    for b in range(_B):
        chunks.append((task, b, 0, b, 0, _T))
        for j in range(_NCH):
            chunks.append((text, b, j * _CR, b, _T + j * _CR, _CR))

    n = len(chunks)

    def in_copy(k):
        src, sb, sr, db, dr, rows = chunks[k]
        return pltpu.make_async_copy(
            src.at[sb, pl.ds(sr, rows), :],
            buf.at[k % _NBUF, pl.ds(0, rows), :], isem.at[k % _NBUF])

    def out_copy(k):
        src, sb, sr, db, dr, rows = chunks[k]
        return pltpu.make_async_copy(
            buf.at[k % _NBUF, pl.ds(0, rows), :],
            oe.at[db, pl.ds(dr, rows), :], osem.at[k % _NBUF])

    for k in range(n + _LAG):
        if k < n:
            if k >= _NBUF:
                out_copy(k - _NBUF).wait()
            in_copy(k).start()
        if k >= _LAG:
            j = k - _LAG
            in_copy(j).wait()
            out_copy(j).start()

    om[...] = jnp.concatenate(
        [jnp.ones((_B, _T), dtype=om.dtype), mask[...]], axis=1)
    ol[...] = jnp.concatenate(
        [jnp.full((_B, _T), -100, dtype=ol.dtype), labels[...]], axis=1)

    for k in range(n - _NBUF, n):
        out_copy(k).wait()


def kernel(text_embeds, attention_mask, labels, task_embeds):
    any_spec = pl.BlockSpec(memory_space=pl.ANY)
    vmem_spec = pl.BlockSpec(memory_space=pltpu.MemorySpace.VMEM)
    return pl.pallas_call(
        _body,
        in_specs=[any_spec, vmem_spec, vmem_spec, any_spec],
        out_specs=[any_spec, vmem_spec, vmem_spec],
        out_shape=(
            jax.ShapeDtypeStruct((_B, _N, _D), jnp.float32),
            jax.ShapeDtypeStruct((_B, _N), jnp.int32),
            jax.ShapeDtypeStruct((_B, _N), jnp.int32),
        ),
        scratch_shapes=[
            pltpu.VMEM((_NBUF, _CR, _D), jnp.float32),
            pltpu.SemaphoreType.DMA((_NBUF,)),
            pltpu.SemaphoreType.DMA((_NBUF,)),
        ],
    )(text_embeds, attention_mask, labels, task_embeds)
